# baseline (speedup 1.0000x reference)
import jax
import jax.numpy as jnp
from jax import lax
from jax.experimental import pallas as pl
from jax.experimental.pallas import tpu as pltpu
from jax.experimental.pallas import tpu_sc as plsc

_MAX_H = 64
_MAX_W = 64
_S = _MAX_H * _MAX_W
_B = 4096
_NC, _NS, _L = 2, 16, 16
_NW = _NC * _NS
_RPT = _B // _NW
_CHUNK = 8
_NCH = _RPT // _CHUNK
_NBUF = 3
_GROUPS = _S // _L


def _sc_body(x_hbm, idx_hbm, pe_hbm, out_hbm,
             pe_raw, pe_s, idx_v, buf,
             sem_in0, sem_in1, sem_in2, sem_out0, sem_out1, sem_out2):
    sem_in = (sem_in0, sem_in1, sem_in2)
    sem_out = (sem_out0, sem_out1, sem_out2)
    wid = lax.axis_index("s") * _NC + lax.axis_index("c")
    base = wid * _RPT

    def in_cp(c, b):
        return pltpu.make_async_copy(
            x_hbm.at[pl.ds(base + c * _CHUNK, _CHUNK)], buf.at[b], sem_in[b])

    def out_cp(c, b):
        return pltpu.make_async_copy(
            buf.at[b], out_hbm.at[pl.ds(base + c * _CHUNK, _CHUNK)],
            sem_out[b])

    for b in range(_NBUF):
        in_cp(b, b).start()

    pltpu.sync_copy(pe_hbm, pe_raw)
    pltpu.sync_copy(idx_hbm, idx_v)

    def gather_body(g, carry):
        s = pl.ds(g * _L, _L)
        vals = plsc.load_gather(pe_raw, [idx_v[s]])
        pe_s[s] = vals * jnp.float32(0.1)
        return carry

    lax.fori_loop(0, _GROUPS, gather_body, 0)

    for c in range(_NCH):
        b = c % _NBUF
        in_cp(c, b).wait()

        def add_body(g, carry, b=b):
            s = pl.ds(g * _L, _L)
            pe_vec = pe_s[s]
            for r in range(_CHUNK):
                plsc.addupdate(buf.at[b, r, s], pe_vec)
            return carry

        lax.fori_loop(0, _GROUPS, add_body, 0)
        out_cp(c, b).start()
        if _NBUF <= c + 1 < _NCH:
            b1 = (c + 1) % _NBUF
            out_cp(c + 1 - _NBUF, b1).wait()
            in_cp(c + 1, b1).start()

    for c in range(_NCH - _NBUF, _NCH):
        out_cp(c, c % _NBUF).wait()


def kernel(x_flat, height, width, pos_embed):
    offset = (jnp.asarray(height, jnp.int32) - _MAX_H) + (
        jnp.asarray(width, jnp.int32) - _MAX_W
    )
    idx = jnp.clip(jnp.arange(_S, dtype=jnp.int32) + offset, 0, _S - 1)
    run = pl.kernel(
        _sc_body,
        out_type=jax.ShapeDtypeStruct((_B, _S), jnp.float32),
        mesh=plsc.VectorSubcoreMesh(core_axis_name="c", subcore_axis_name="s"),
        compiler_params=pltpu.CompilerParams(needs_layout_passes=False),
        scratch_types=[
            pltpu.VMEM((_S,), jnp.float32),
            pltpu.VMEM((_S,), jnp.float32),
            pltpu.VMEM((_S,), jnp.int32),
            pltpu.VMEM((_NBUF, _CHUNK, _S), jnp.float32),
            pltpu.SemaphoreType.DMA,
            pltpu.SemaphoreType.DMA,
            pltpu.SemaphoreType.DMA,
            pltpu.SemaphoreType.DMA,
            pltpu.SemaphoreType.DMA,
            pltpu.SemaphoreType.DMA,
        ],
    )
    return run(x_flat, idx, pos_embed)


# speedup vs baseline: 1.1837x; 1.1837x over previous
import jax
import jax.numpy as jnp
from jax import lax
from jax.experimental import pallas as pl
from jax.experimental.pallas import tpu as pltpu
from jax.experimental.pallas import tpu_sc as plsc

_MAX_H = 64
_MAX_W = 64
_S = _MAX_H * _MAX_W
_B = 4096
_NC, _NS, _L = 2, 16, 16
_NW = _NC * _NS
_RPT = _B // _NW
_CHUNK = 4
_NCH = _RPT // _CHUNK
_NBUF = 3
_GROUPS = _S // _L


def _sc_body(x_hbm, idx_hbm, pe_hbm, out_hbm,
             pe_raw, pe_s, idx_v, buf_in, buf_out,
             sem_in0, sem_in1, sem_in2, sem_out0, sem_out1, sem_out2):
    sem_in = (sem_in0, sem_in1, sem_in2)
    sem_out = (sem_out0, sem_out1, sem_out2)
    wid = lax.axis_index("s") * _NC + lax.axis_index("c")
    base = wid * _RPT

    def in_cp(c, b):
        return pltpu.make_async_copy(
            x_hbm.at[pl.ds(base + c * _CHUNK, _CHUNK)], buf_in.at[b], sem_in[b])

    def out_cp(c, b):
        return pltpu.make_async_copy(
            buf_out.at[b], out_hbm.at[pl.ds(base + c * _CHUNK, _CHUNK)],
            sem_out[b])

    for b in range(_NBUF):
        in_cp(b, b).start()

    pltpu.sync_copy(pe_hbm, pe_raw)
    pltpu.sync_copy(idx_hbm, idx_v)

    def gather_body(g, carry):
        s = pl.ds(g * _L, _L)
        vals = plsc.load_gather(pe_raw, [idx_v[s]])
        pe_s[s] = vals * jnp.float32(0.1)
        return carry

    lax.fori_loop(0, _GROUPS, gather_body, 0)

    for c in range(_NCH):
        b = c % _NBUF
        in_cp(c, b).wait()
        if c >= _NBUF:
            out_cp(c - _NBUF, b).wait()
        bi = buf_in.at[b]
        bo = buf_out.at[b]

        def add_body(g, carry, bi=bi, bo=bo):
            s = pl.ds(g * _L, _L)
            pe_vec = pe_s[s]
            for r in range(_CHUNK):
                bo[r, s] = bi[r, s] + pe_vec
            return carry

        lax.fori_loop(0, _GROUPS, add_body, 0)
        out_cp(c, b).start()
        if c + _NBUF < _NCH:
            in_cp(c + _NBUF, b).start()

    for c in range(_NCH - _NBUF, _NCH):
        out_cp(c, c % _NBUF).wait()


def kernel(x_flat, height, width, pos_embed):
    offset = (jnp.asarray(height, jnp.int32) - _MAX_H) + (
        jnp.asarray(width, jnp.int32) - _MAX_W
    )
    idx = jnp.clip(jnp.arange(_S, dtype=jnp.int32) + offset, 0, _S - 1)
    run = pl.kernel(
        _sc_body,
        out_type=jax.ShapeDtypeStruct((_B, _S), jnp.float32),
        mesh=plsc.VectorSubcoreMesh(core_axis_name="c", subcore_axis_name="s"),
        compiler_params=pltpu.CompilerParams(needs_layout_passes=False),
        scratch_types=[
            pltpu.VMEM((_S,), jnp.float32),
            pltpu.VMEM((_S,), jnp.float32),
            pltpu.VMEM((_S,), jnp.int32),
            pltpu.VMEM((_NBUF, _CHUNK, _S), jnp.float32),
            pltpu.VMEM((_NBUF, _CHUNK, _S), jnp.float32),
            pltpu.SemaphoreType.DMA,
            pltpu.SemaphoreType.DMA,
            pltpu.SemaphoreType.DMA,
            pltpu.SemaphoreType.DMA,
            pltpu.SemaphoreType.DMA,
            pltpu.SemaphoreType.DMA,
        ],
    )
    return run(x_flat, idx, pos_embed)


# speedup vs baseline: 1.2296x; 1.0387x over previous
import jax
import jax.numpy as jnp
from jax import lax
from jax.experimental import pallas as pl
from jax.experimental.pallas import tpu as pltpu
from jax.experimental.pallas import tpu_sc as plsc

_MAX_H = 64
_MAX_W = 64
_S = _MAX_H * _MAX_W
_B = 4096
_NC, _NS, _L = 2, 16, 16
_NW = _NC * _NS
_RPT = _B // _NW
_CHUNK = 4
_NCH = _RPT // _CHUNK
_NBUF = 3
_GROUPS = _S // _L


def _sc_body(x_hbm, off_hbm, pe_hbm, out_hbm,
             pe_raw, pe_s, off_v, buf_in, buf_out, sem_in, sem_out):
    wid = lax.axis_index("s") * _NC + lax.axis_index("c")
    base = wid * _RPT

    def in_cp(c, b):
        return pltpu.make_async_copy(
            x_hbm.at[pl.ds(base + c * _CHUNK, _CHUNK)], buf_in.at[b],
            sem_in.at[b])

    def out_cp(c, b):
        return pltpu.make_async_copy(
            buf_out.at[b], out_hbm.at[pl.ds(base + c * _CHUNK, _CHUNK)],
            sem_out.at[b])

    for b in range(_NBUF):
        in_cp(b, b).start()

    pltpu.sync_copy(pe_hbm, pe_raw)
    pltpu.sync_copy(off_hbm, off_v)
    lanes = lax.iota(jnp.int32, _L)
    off = off_v[...]

    def gather_body(g, carry):
        s = pl.ds(g * _L, _L)
        iv = jnp.clip(lanes + (g * _L + off), 0, _S - 1)
        pe_s[s] = plsc.load_gather(pe_raw, [iv]) * jnp.float32(0.1)
        return carry

    lax.fori_loop(0, _GROUPS, gather_body, 0)

    def chunk_body(c, carry):
        b = lax.rem(c, _NBUF)
        in_cp(c, b).wait()

        @pl.when(c >= _NBUF)
        def _():
            out_cp(c - _NBUF, b).wait()

        def add_body(g, carry2):
            s = pl.ds(g * _L, _L)
            pe_vec = pe_s[s]
            for r in range(_CHUNK):
                buf_out[b, r, s] = buf_in[b, r, s] + pe_vec
            return carry2

        lax.fori_loop(0, _GROUPS, add_body, 0)
        out_cp(c, b).start()

        @pl.when(c + _NBUF < _NCH)
        def _():
            in_cp(c + _NBUF, b).start()

        return carry

    lax.fori_loop(0, _NCH, chunk_body, 0)

    def drain_body(c, carry):
        out_cp(c, lax.rem(c, _NBUF)).wait()
        return carry

    lax.fori_loop(_NCH - _NBUF, _NCH, drain_body, 0)


def kernel(x_flat, height, width, pos_embed):
    offset = (jnp.asarray(height, jnp.int32) - _MAX_H) + (
        jnp.asarray(width, jnp.int32) - _MAX_W
    )
    off_vec = jnp.full((_L,), offset, dtype=jnp.int32)
    run = pl.kernel(
        _sc_body,
        out_type=jax.ShapeDtypeStruct((_B, _S), jnp.float32),
        mesh=plsc.VectorSubcoreMesh(core_axis_name="c", subcore_axis_name="s"),
        compiler_params=pltpu.CompilerParams(needs_layout_passes=False),
        scratch_types=[
            pltpu.VMEM((_S,), jnp.float32),
            pltpu.VMEM((_S,), jnp.float32),
            pltpu.VMEM((_L,), jnp.int32),
            pltpu.VMEM((_NBUF, _CHUNK, _S), jnp.float32),
            pltpu.VMEM((_NBUF, _CHUNK, _S), jnp.float32),
            pltpu.SemaphoreType.DMA((_NBUF,)),
            pltpu.SemaphoreType.DMA((_NBUF,)),
        ],
    )
    return run(x_flat, off_vec, pos_embed)


# speedup vs baseline: 1.4085x; 1.1455x over previous
import jax
import jax.numpy as jnp
from jax import lax
from jax.experimental import pallas as pl
from jax.experimental.pallas import tpu as pltpu
from jax.experimental.pallas import tpu_sc as plsc

_MAX_H = 64
_MAX_W = 64
_S = _MAX_H * _MAX_W
_B = 4096
_NC, _NS, _L = 2, 16, 16
_NW = _NC * _NS
_RPT = _B // _NW
_CHUNK = 4
_NCH = _RPT // _CHUNK
_NBUF = 3
_GROUPS = _S // _L


def _sc_body(x_hbm, off_hbm, pe_hbm, out_hbm,
             pe_raw, pe_s, off_v, buf_in, buf_out, sem_in, sem_out):
    wid = lax.axis_index("s") * _NC + lax.axis_index("c")
    base = wid * _RPT

    def in_cp(c, b):
        return pltpu.make_async_copy(
            x_hbm.at[pl.ds(base + c * _CHUNK, _CHUNK)], buf_in.at[b],
            sem_in.at[b])

    def out_cp(c, b):
        return pltpu.make_async_copy(
            buf_out.at[b], out_hbm.at[pl.ds(base + c * _CHUNK, _CHUNK)],
            sem_out.at[b])

    for b in range(_NBUF):
        in_cp(b, b).start()

    pltpu.sync_copy(pe_hbm, pe_raw)
    pltpu.sync_copy(off_hbm, off_v)
    lanes = lax.iota(jnp.int32, _L)
    off = off_v[...]

    def gather_body(g, carry):
        s = pl.ds(g * _L, _L)
        iv = jnp.clip(lanes + (g * _L + off), 0, _S - 1)
        pe_s[s] = plsc.load_gather(pe_raw, [iv]) * jnp.float32(0.1)
        return carry

    lax.fori_loop(0, _GROUPS, gather_body, 0)

    def chunk_body(c, carry):
        b = lax.rem(c, _NBUF)
        in_cp(c, b).wait()

        @pl.when(c >= _NBUF)
        def _():
            out_cp(c - _NBUF, b).wait()

        @plsc.parallel_loop(0, _GROUPS, unroll=2)
        def add_body(g):
            s = pl.ds(g * _L, _L)
            pe_vec = pe_s[s]
            for r in range(_CHUNK):
                buf_out[b, r, s] = buf_in[b, r, s] + pe_vec
        out_cp(c, b).start()

        @pl.when(c + _NBUF < _NCH)
        def _():
            in_cp(c + _NBUF, b).start()

        return carry

    lax.fori_loop(0, _NCH, chunk_body, 0)

    def drain_body(c, carry):
        out_cp(c, lax.rem(c, _NBUF)).wait()
        return carry

    lax.fori_loop(_NCH - _NBUF, _NCH, drain_body, 0)


def kernel(x_flat, height, width, pos_embed):
    offset = (jnp.asarray(height, jnp.int32) - _MAX_H) + (
        jnp.asarray(width, jnp.int32) - _MAX_W
    )
    off_vec = jnp.full((_L,), offset, dtype=jnp.int32)
    run = pl.kernel(
        _sc_body,
        out_type=jax.ShapeDtypeStruct((_B, _S), jnp.float32),
        mesh=plsc.VectorSubcoreMesh(core_axis_name="c", subcore_axis_name="s"),
        compiler_params=pltpu.CompilerParams(needs_layout_passes=False),
        scratch_types=[
            pltpu.VMEM((_S,), jnp.float32),
            pltpu.VMEM((_S,), jnp.float32),
            pltpu.VMEM((_L,), jnp.int32),
            pltpu.VMEM((_NBUF, _CHUNK, _S), jnp.float32),
            pltpu.VMEM((_NBUF, _CHUNK, _S), jnp.float32),
            pltpu.SemaphoreType.DMA((_NBUF,)),
            pltpu.SemaphoreType.DMA((_NBUF,)),
        ],
    )
    return run(x_flat, off_vec, pos_embed)


# speedup vs baseline: 1.4105x; 1.0015x over previous
import jax
import jax.numpy as jnp
from jax import lax
from jax.experimental import pallas as pl
from jax.experimental.pallas import tpu as pltpu
from jax.experimental.pallas import tpu_sc as plsc

_MAX_H = 64
_MAX_W = 64
_S = _MAX_H * _MAX_W
_B = 4096
_NC, _NS, _L = 2, 16, 16
_NW = _NC * _NS
_RPT = _B // _NW
_CHUNK = 4
_NCH = _RPT // _CHUNK
_NBUF = 3
_GROUPS = _S // _L


def _sc_body(x_hbm, off_hbm, pe_hbm, out_hbm,
             pe_raw, pe_s, off_v, buf_in, buf_out, sem_in, sem_out):
    wid = lax.axis_index("s") * _NC + lax.axis_index("c")
    base = wid * _RPT

    def in_cp(c, b):
        return pltpu.make_async_copy(
            x_hbm.at[pl.ds(base + c * _CHUNK, _CHUNK)], buf_in.at[b],
            sem_in.at[b])

    def out_cp(c, b):
        return pltpu.make_async_copy(
            buf_out.at[b], out_hbm.at[pl.ds(base + c * _CHUNK, _CHUNK)],
            sem_out.at[b])

    for b in range(_NBUF):
        in_cp(b, b).start()

    pltpu.sync_copy(pe_hbm, pe_raw)
    pltpu.sync_copy(off_hbm, off_v)
    lanes = lax.iota(jnp.int32, _L)
    off = off_v[...]

    @plsc.parallel_loop(0, _GROUPS, unroll=2)
    def gather_body(g):
        s = pl.ds(g * _L, _L)
        iv = jnp.clip(lanes + (g * _L + off), 0, _S - 1)
        pe_s[s] = plsc.load_gather(pe_raw, [iv]) * jnp.float32(0.1)

    def chunk_body(c, carry):
        b = lax.rem(c, _NBUF)
        in_cp(c, b).wait()

        @pl.when(c >= _NBUF)
        def _():
            out_cp(c - _NBUF, b).wait()

        @plsc.parallel_loop(0, _GROUPS, unroll=4)
        def add_body(g):
            s = pl.ds(g * _L, _L)
            pe_vec = pe_s[s]
            for r in range(_CHUNK):
                buf_out[b, r, s] = buf_in[b, r, s] + pe_vec
        out_cp(c, b).start()

        @pl.when(c + _NBUF < _NCH)
        def _():
            in_cp(c + _NBUF, b).start()

        return carry

    lax.fori_loop(0, _NCH, chunk_body, 0)

    def drain_body(c, carry):
        out_cp(c, lax.rem(c, _NBUF)).wait()
        return carry

    lax.fori_loop(_NCH - _NBUF, _NCH, drain_body, 0)


def kernel(x_flat, height, width, pos_embed):
    offset = (jnp.asarray(height, jnp.int32) - _MAX_H) + (
        jnp.asarray(width, jnp.int32) - _MAX_W
    )
    off_vec = jnp.full((_L,), offset, dtype=jnp.int32)
    run = pl.kernel(
        _sc_body,
        out_type=jax.ShapeDtypeStruct((_B, _S), jnp.float32),
        mesh=plsc.VectorSubcoreMesh(core_axis_name="c", subcore_axis_name="s"),
        compiler_params=pltpu.CompilerParams(needs_layout_passes=False),
        scratch_types=[
            pltpu.VMEM((_S,), jnp.float32),
            pltpu.VMEM((_S,), jnp.float32),
            pltpu.VMEM((_L,), jnp.int32),
            pltpu.VMEM((_NBUF, _CHUNK, _S), jnp.float32),
            pltpu.VMEM((_NBUF, _CHUNK, _S), jnp.float32),
            pltpu.SemaphoreType.DMA((_NBUF,)),
            pltpu.SemaphoreType.DMA((_NBUF,)),
        ],
    )
    return run(x_flat, off_vec, pos_embed)
